# Initial kernel scaffold; baseline (speedup 1.0000x reference)
import jax
import jax.numpy as jnp
from jax.experimental import pallas as pl


def kernel(idx, embedding):
    raise NotImplementedError("write your pallas kernel here")


# speedup vs baseline: 1.0220x; 1.0220x over previous
import functools

import jax
import jax.numpy as jnp
from jax import lax
from jax.experimental import pallas as pl
from jax.experimental.pallas import tpu as pltpu
from jax.experimental.pallas import tpu_sc as plsc

_NUM_CORES = 2
_NUM_SUBCORES = 16
_NUM_WORKERS = _NUM_CORES * _NUM_SUBCORES
_CHUNK = 128


@functools.cache
def _build_gather(num_rows: int, feat: int):
    chunks_total = num_rows // _CHUNK
    chunks_per_w = chunks_total // _NUM_WORKERS

    mesh = plsc.VectorSubcoreMesh(core_axis_name="c", subcore_axis_name="s")

    @functools.partial(
        pl.kernel,
        mesh=mesh,
        out_type=jax.ShapeDtypeStruct((num_rows, feat), jnp.float32),
        scratch_types=[
            pltpu.VMEM((chunks_per_w, _CHUNK), jnp.int32),
            pltpu.VMEM((_CHUNK, feat), jnp.float32),
            pltpu.SemaphoreType.DMA,
        ],
        compiler_params=pltpu.CompilerParams(use_tc_tiling_on_sc=False),
    )
    def gather_kernel(idx_hbm, table_hbm, out_hbm, idx_v, rows_v, sem):
        wid = lax.axis_index("s") * _NUM_CORES + lax.axis_index("c")
        chunk0 = wid * chunks_per_w
        pltpu.sync_copy(idx_hbm.at[pl.ds(chunk0, chunks_per_w)], idx_v)

        def step(j, carry):
            pltpu.async_copy(table_hbm.at[idx_v.at[j]], rows_v, sem).wait()
            pltpu.sync_copy(
                rows_v, out_hbm.at[pl.ds((chunk0 + j) * _CHUNK, _CHUNK)]
            )
            return carry

        lax.fori_loop(0, chunks_per_w, step, 0)

    return gather_kernel


def kernel(idx, embedding):
    batch, hist = idx.shape
    num_rows = batch * hist
    feat = embedding.shape[1]
    idx2d = idx.astype(jnp.int32).reshape(num_rows // _CHUNK, _CHUNK)
    out = _build_gather(num_rows, feat)(idx2d, embedding)
    return out.reshape(batch, hist, feat)


# speedup vs baseline: 1.1125x; 1.0885x over previous
import functools

import jax
import jax.numpy as jnp
from jax import lax
from jax.experimental import pallas as pl
from jax.experimental.pallas import tpu as pltpu
from jax.experimental.pallas import tpu_sc as plsc

_NUM_CORES = 2
_NUM_SUBCORES = 16
_NUM_WORKERS = _NUM_CORES * _NUM_SUBCORES
_CHUNK = 128


_K = 8


@functools.cache
def _build_gather(num_rows: int, feat: int):
    chunks_total = num_rows // _CHUNK
    chunks_per_w = chunks_total // _NUM_WORKERS
    groups = chunks_per_w // _K
    grows = _K * _CHUNK

    mesh = plsc.VectorSubcoreMesh(core_axis_name="c", subcore_axis_name="s")

    @functools.partial(
        pl.kernel,
        mesh=mesh,
        out_type=jax.ShapeDtypeStruct((num_rows, feat), jnp.float32),
        scratch_types=[
            pltpu.VMEM((chunks_per_w, _CHUNK), jnp.int32),
            pltpu.VMEM((2, grows, feat), jnp.float32),
            pltpu.SemaphoreType.DMA((2,)),
            pltpu.SemaphoreType.DMA((2,)),
        ],
        compiler_params=pltpu.CompilerParams(use_tc_tiling_on_sc=False),
    )
    def gather_kernel(idx_hbm, table_hbm, out_hbm, idx_v, bufs, gsem, ssem):
        wid = lax.axis_index("s") * _NUM_CORES + lax.axis_index("c")
        chunk0 = wid * chunks_per_w
        row0 = chunk0 * _CHUNK
        pltpu.sync_copy(idx_hbm.at[pl.ds(chunk0, chunks_per_w)], idx_v)

        def fire(g, p):
            for t in range(_K):
                pltpu.async_copy(
                    table_hbm.at[idx_v.at[g * _K + t]],
                    bufs.at[p, pl.ds(t * _CHUNK, _CHUNK)],
                    gsem.at[p],
                )

        def drain_gathers(p):
            pltpu.make_async_copy(
                table_hbm.at[pl.ds(0, grows)], bufs.at[p], gsem.at[p]
            ).wait()

        def store(g, p):
            pltpu.async_copy(
                bufs.at[p],
                out_hbm.at[pl.ds(row0 + g * grows, grows)],
                ssem.at[p],
            )

        def wait_store(p):
            pltpu.make_async_copy(
                bufs.at[p], out_hbm.at[pl.ds(row0, grows)], ssem.at[p]
            ).wait()

        fire(0, 0)

        def body(g, carry):
            p = lax.rem(g, 2)
            q = 1 - p

            @pl.when(g >= 2)
            def _():
                wait_store(p)

            fire(g, p)
            drain_gathers(q)
            store(g - 1, q)
            return carry

        lax.fori_loop(1, groups, body, 0)

        p_last = (groups - 1) % 2
        drain_gathers(p_last)
        if groups >= 2:
            wait_store(1 - p_last)
        store(groups - 1, p_last)
        wait_store(p_last)

    return gather_kernel


def kernel(idx, embedding):
    batch, hist = idx.shape
    num_rows = batch * hist
    feat = embedding.shape[1]
    idx2d = idx.astype(jnp.int32).reshape(num_rows // _CHUNK, _CHUNK)
    out = _build_gather(num_rows, feat)(idx2d, embedding)
    return out.reshape(batch, hist, feat)


# speedup vs baseline: 1.6455x; 1.4791x over previous
import functools

import jax
import jax.numpy as jnp
from jax import lax
from jax.experimental import pallas as pl
from jax.experimental.pallas import tpu as pltpu
from jax.experimental.pallas import tpu_sc as plsc

_NUM_CORES = 2
_NUM_SUBCORES = 16
_NUM_WORKERS = _NUM_CORES * _NUM_SUBCORES
_CHUNK = 128
_LANES = 16


@functools.cache
def _build_gather(num_rows: int, feat: int, hist: int):
    batch = num_rows // hist
    btiles = batch // _CHUNK
    ftiles = feat // 8
    chunks_total = num_rows // _CHUNK
    chunks_per_w = chunks_total // _NUM_WORKERS
    rows_per_w = chunks_per_w * _CHUNK

    mesh = plsc.VectorSubcoreMesh(core_axis_name="c", subcore_axis_name="s")

    @functools.partial(
        pl.kernel,
        mesh=mesh,
        out_type=jax.ShapeDtypeStruct(
            (hist, ftiles, btiles, 8, _CHUNK), jnp.float32
        ),
        scratch_types=[
            pltpu.VMEM((rows_per_w,), jnp.int32),
            pltpu.VMEM((2, _CHUNK, feat), jnp.float32),
            pltpu.VMEM((2, ftiles, 8, _CHUNK), jnp.float32),
            pltpu.SemaphoreType.DMA((2,)),
            pltpu.SemaphoreType.DMA((2,)),
        ],
        compiler_params=pltpu.CompilerParams(
            use_tc_tiling_on_sc=False, needs_layout_passes=False
        ),
    )
    def gather_kernel(idx_hbm, table_hbm, out_hbm, idx_v, rows, tbuf, gsem, ssem):
        wid = lax.axis_index("s") * _NUM_CORES + lax.axis_index("c")
        chunk0 = wid * chunks_per_w
        pltpu.sync_copy(idx_hbm.at[pl.ds(chunk0 * _CHUNK, rows_per_w)], idx_v)

        lane = lax.iota(jnp.int32, _LANES)

        def fire_gather(j, s):
            pltpu.async_copy(
                table_hbm.at[idx_v.at[pl.ds(j * _CHUNK, _CHUNK)]],
                rows.at[s],
                gsem.at[s],
            )

        def drain_gather(s):
            pltpu.make_async_copy(
                table_hbm.at[pl.ds(0, _CHUNK)], rows.at[s], gsem.at[s]
            ).wait()

        def wait_stores(s):
            pltpu.make_async_copy(
                tbuf.at[s], out_hbm.at[0, pl.ds(0, ftiles), 0], ssem.at[s]
            ).wait()

        def transpose(s):
            for ft in range(ftiles):
                for fi in range(8):
                    f = 8 * ft + fi
                    col = jnp.full((_LANES,), f, jnp.int32)
                    for k in range(_CHUNK // _LANES):
                        rvec = plsc.load_gather(
                            rows.at[s], [lane + (k * _LANES), col]
                        )
                        tbuf[s, ft, fi, pl.ds(k * _LANES, _LANES)] = rvec

        def fire_stores(j, s):
            c = chunk0 + j
            h = c // btiles
            bb = lax.rem(c, btiles)
            for ft in range(ftiles):
                pltpu.async_copy(
                    tbuf.at[s, ft], out_hbm.at[h, ft, bb], ssem.at[s]
                )

        fire_gather(0, 0)
        fire_gather(1, 1)

        def body(j, carry):
            s = lax.rem(j, 2)
            drain_gather(s)

            @pl.when(j >= 2)
            def _():
                wait_stores(s)

            transpose(s)

            @pl.when(j + 2 < chunks_per_w)
            def _():
                fire_gather(j + 2, s)

            fire_stores(j, s)
            return carry

        lax.fori_loop(0, chunks_per_w, body, 0)
        wait_stores((chunks_per_w - 2) % 2)
        wait_stores((chunks_per_w - 1) % 2)

    return gather_kernel


def kernel(idx, embedding):
    batch, hist = idx.shape
    feat = embedding.shape[1]
    idx1 = idx.T.reshape(-1).astype(jnp.int32)
    out5 = _build_gather(batch * hist, feat, hist)(idx1, embedding)
    return out5.transpose(2, 4, 0, 1, 3).reshape(batch, hist, feat)


# speedup vs baseline: 1.9515x; 1.1860x over previous
import functools

import jax
import jax.numpy as jnp
from jax import lax
from jax.experimental import pallas as pl
from jax.experimental.pallas import tpu as pltpu
from jax.experimental.pallas import tpu_sc as plsc

_NUM_CORES = 2
_NUM_SUBCORES = 16
_NUM_WORKERS = _NUM_CORES * _NUM_SUBCORES
_CHUNK = 128
_LANES = 16


@functools.cache
def _build_gather(num_rows: int, feat: int, hist: int):
    batch = num_rows // hist
    btiles = batch // _CHUNK
    ftiles = feat // 8
    chunks_total = num_rows // _CHUNK
    chunks_per_w = chunks_total // _NUM_WORKERS
    rows_per_w = chunks_per_w * _CHUNK

    mesh = plsc.VectorSubcoreMesh(core_axis_name="c", subcore_axis_name="s")

    @functools.partial(
        pl.kernel,
        mesh=mesh,
        out_type=jax.ShapeDtypeStruct(
            (hist, ftiles, btiles, 8, _CHUNK), jnp.float32
        ),
        scratch_types=[
            pltpu.VMEM((rows_per_w,), jnp.int32),
            pltpu.VMEM((2, _CHUNK, feat), jnp.float32),
            pltpu.VMEM((2, feat, _CHUNK), jnp.float32),
            pltpu.SemaphoreType.DMA((2,)),
            pltpu.SemaphoreType.DMA((2,)),
        ],
        compiler_params=pltpu.CompilerParams(
            use_tc_tiling_on_sc=False, needs_layout_passes=False
        ),
    )
    def gather_kernel(idx_hbm, table_hbm, out_hbm, idx_v, rows, tbuf, gsem, ssem):
        wid = lax.axis_index("s") * _NUM_CORES + lax.axis_index("c")
        chunk0 = wid * chunks_per_w
        pltpu.sync_copy(idx_hbm.at[pl.ds(chunk0 * _CHUNK, rows_per_w)], idx_v)

        lane = lax.iota(jnp.int32, _LANES)
        fvecs = [
            [
                lax.rem(lane + j, _LANES) + _LANES * kf
                for j in range(_LANES)
            ]
            for kf in range(feat // _LANES)
        ]
        bvecs = [lane + _LANES * kb for kb in range(_CHUNK // _LANES)]

        def fire_gather(j, s):
            pltpu.async_copy(
                table_hbm.at[idx_v.at[pl.ds(j * _CHUNK, _CHUNK)]],
                rows.at[s],
                gsem.at[s],
            )

        def drain_gather(s):
            pltpu.make_async_copy(
                table_hbm.at[pl.ds(0, _CHUNK)], rows.at[s], gsem.at[s]
            ).wait()

        def wait_stores(s):
            for ft in range(ftiles):
                pltpu.make_async_copy(
                    tbuf.at[s, pl.ds(ft * 8, 8)],
                    out_hbm.at[0, ft, 0],
                    ssem.at[s],
                ).wait()

        def transpose(s):
            for kb in range(_CHUNK // _LANES):
                bv = bvecs[kb]
                for kf in range(feat // _LANES):
                    for j in range(_LANES):
                        fv = fvecs[kf][j]
                        rvec = plsc.load_gather(rows.at[s], [bv, fv])
                        plsc.store_scatter(tbuf.at[s], [fv, bv], rvec)

        def fire_stores(j, s):
            c = chunk0 + j
            h = c // btiles
            bb = lax.rem(c, btiles)
            for ft in range(ftiles):
                pltpu.async_copy(
                    tbuf.at[s, pl.ds(ft * 8, 8)],
                    out_hbm.at[h, ft, bb],
                    ssem.at[s],
                )

        fire_gather(0, 0)
        fire_gather(1, 1)

        def body(j, carry):
            s = lax.rem(j, 2)
            drain_gather(s)

            @pl.when(j >= 2)
            def _():
                wait_stores(s)

            transpose(s)

            @pl.when(j + 2 < chunks_per_w)
            def _():
                fire_gather(j + 2, s)

            fire_stores(j, s)
            return carry

        lax.fori_loop(0, chunks_per_w, body, 0)
        wait_stores((chunks_per_w - 2) % 2)
        wait_stores((chunks_per_w - 1) % 2)

    return gather_kernel


def kernel(idx, embedding):
    batch, hist = idx.shape
    feat = embedding.shape[1]
    idx1 = idx.T.reshape(-1).astype(jnp.int32)
    out5 = _build_gather(batch * hist, feat, hist)(idx1, embedding)
    return out5.transpose(2, 4, 0, 1, 3).reshape(batch, hist, feat)


# speedup vs baseline: 2.6259x; 1.3455x over previous
import functools

import jax
import jax.numpy as jnp
from jax import lax
from jax.experimental import pallas as pl
from jax.experimental.pallas import tpu as pltpu
from jax.experimental.pallas import tpu_sc as plsc

_NUM_CORES = 2
_NUM_SUBCORES = 16
_NUM_WORKERS = _NUM_CORES * _NUM_SUBCORES
_CHUNK = 128
_LANES = 16


@functools.cache
def _build_gather(num_rows: int, feat: int, hist: int):
    batch = num_rows // hist
    btiles = batch // _CHUNK
    ftiles = feat // 8
    chunks_total = num_rows // _CHUNK
    chunks_per_w = chunks_total // _NUM_WORKERS
    rows_per_w = chunks_per_w * _CHUNK

    mesh = plsc.VectorSubcoreMesh(core_axis_name="c", subcore_axis_name="s")

    @functools.partial(
        pl.kernel,
        mesh=mesh,
        out_type=jax.ShapeDtypeStruct(
            (hist, ftiles, btiles, 8, _CHUNK), jnp.float32
        ),
        scratch_types=[
            pltpu.VMEM((rows_per_w,), jnp.int32),
            pltpu.VMEM((2, _CHUNK, feat), jnp.float32),
            pltpu.VMEM((2, feat, _CHUNK), jnp.float32),
            pltpu.SemaphoreType.DMA((2,)),
            pltpu.SemaphoreType.DMA((2,)),
        ],
        compiler_params=pltpu.CompilerParams(
            use_tc_tiling_on_sc=False, needs_layout_passes=False
        ),
    )
    def gather_kernel(idx_hbm, table_hbm, out_hbm, idx_v, rows, tbuf, gsem, ssem):
        wid = lax.axis_index("s") * _NUM_CORES + lax.axis_index("c")
        chunk0 = wid * chunks_per_w
        pltpu.sync_copy(idx_hbm.at[pl.ds(chunk0 * _CHUNK, rows_per_w)], idx_v)

        lane = lax.iota(jnp.int32, _LANES)
        fvecs = [lax.rem(lane + j, _LANES) for j in range(_LANES)]

        def fire_gather(j, s):
            pltpu.async_copy(
                table_hbm.at[idx_v.at[pl.ds(j * _CHUNK, _CHUNK)]],
                rows.at[s],
                gsem.at[s],
            )

        def drain_gather(s):
            pltpu.make_async_copy(
                table_hbm.at[pl.ds(0, _CHUNK)], rows.at[s], gsem.at[s]
            ).wait()

        def wait_stores(s):
            for ft in range(ftiles):
                pltpu.make_async_copy(
                    tbuf.at[s, pl.ds(ft * 8, 8)],
                    out_hbm.at[0, ft, 0],
                    ssem.at[s],
                ).wait()

        def transpose(s):
            def kb_body(kb, carry):
                bv = lane + kb * _LANES
                for j in range(_LANES):
                    fv = fvecs[j]
                    for kf in range(feat // _LANES):
                        fvk = fv + _LANES * kf if kf else fv
                        rvec = plsc.load_gather(rows.at[s], [bv, fvk])
                        plsc.store_scatter(tbuf.at[s], [fvk, bv], rvec)
                return carry

            lax.fori_loop(0, _CHUNK // _LANES, kb_body, 0)

        def fire_stores(j, s):
            c = chunk0 + j
            h = c // btiles
            bb = lax.rem(c, btiles)
            for ft in range(ftiles):
                pltpu.async_copy(
                    tbuf.at[s, pl.ds(ft * 8, 8)],
                    out_hbm.at[h, ft, bb],
                    ssem.at[s],
                )

        fire_gather(0, 0)
        fire_gather(1, 1)

        def body(g, carry):
            for s in (0, 1):
                j = 2 * g + s
                drain_gather(s)

                @pl.when(g >= 1)
                def _():
                    wait_stores(s)

                transpose(s)

                @pl.when(g <= chunks_per_w // 2 - 2)
                def _():
                    fire_gather(j + 2, s)

                fire_stores(j, s)
            return carry

        lax.fori_loop(0, chunks_per_w // 2, body, 0)
        wait_stores(0)
        wait_stores(1)

    return gather_kernel


def kernel(idx, embedding):
    batch, hist = idx.shape
    feat = embedding.shape[1]
    idx1 = idx.T.reshape(-1).astype(jnp.int32)
    out5 = _build_gather(batch * hist, feat, hist)(idx1, embedding)
    return out5.transpose(2, 4, 0, 1, 3).reshape(batch, hist, feat)


# speedup vs baseline: 2.8572x; 1.0881x over previous
import functools

import jax
import jax.numpy as jnp
from jax import lax
from jax.experimental import pallas as pl
from jax.experimental.pallas import tpu as pltpu
from jax.experimental.pallas import tpu_sc as plsc

_NUM_CORES = 2
_NUM_SUBCORES = 16
_NUM_WORKERS = _NUM_CORES * _NUM_SUBCORES
_CHUNK = 128
_LANES = 16


@functools.cache
def _build_gather(num_rows: int, feat: int, hist: int):
    batch = num_rows // hist
    btiles = batch // _CHUNK
    ftiles = feat // 8
    chunks_total = num_rows // _CHUNK
    chunks_per_w = chunks_total // _NUM_WORKERS
    rows_per_w = chunks_per_w * _CHUNK

    mesh = plsc.VectorSubcoreMesh(core_axis_name="c", subcore_axis_name="s")

    @functools.partial(
        pl.kernel,
        mesh=mesh,
        out_type=jax.ShapeDtypeStruct(
            (hist, ftiles, btiles, 8, _CHUNK), jnp.float32
        ),
        scratch_types=[
            pltpu.VMEM((rows_per_w,), jnp.int32),
            pltpu.VMEM((2, _CHUNK, feat), jnp.float32),
            pltpu.VMEM((2, feat, _CHUNK), jnp.float32),
            pltpu.SemaphoreType.DMA((2,)),
            pltpu.SemaphoreType.DMA((2,)),
        ],
        compiler_params=pltpu.CompilerParams(
            use_tc_tiling_on_sc=False, needs_layout_passes=False
        ),
    )
    def gather_kernel(idx_hbm, table_hbm, out_hbm, idx_v, rows, tbuf, gsem, ssem):
        wid = lax.axis_index("s") * _NUM_CORES + lax.axis_index("c")
        chunk0 = wid * chunks_per_w
        pltpu.sync_copy(idx_hbm.at[pl.ds(chunk0 * _CHUNK, rows_per_w)], idx_v)

        lane = lax.iota(jnp.int32, _LANES)
        fvecs = [lax.rem(lane + j, _LANES) for j in range(_LANES)]

        def fire_gather(j, s):
            pltpu.async_copy(
                table_hbm.at[idx_v.at[pl.ds(j * _CHUNK, _CHUNK)]],
                rows.at[s],
                gsem.at[s],
            )

        def drain_gather(s):
            pltpu.make_async_copy(
                table_hbm.at[pl.ds(0, _CHUNK)], rows.at[s], gsem.at[s]
            ).wait()

        def wait_stores(s):
            for ft in range(ftiles):
                pltpu.make_async_copy(
                    tbuf.at[s, pl.ds(ft * 8, 8)],
                    out_hbm.at[0, ft, 0],
                    ssem.at[s],
                ).wait()

        def transpose(s):
            def kb_body(kb, carry):
                for u in range(2):
                    bv = lane + (kb * 2 + u) * _LANES
                    for j0 in range(0, _LANES, 4):
                        vals = []
                        for j in range(j0, j0 + 4):
                            fv = fvecs[j]
                            for kf in range(feat // _LANES):
                                fvk = fv + _LANES * kf if kf else fv
                                vals.append(
                                    (fvk, plsc.load_gather(rows.at[s], [bv, fvk]))
                                )
                        for fvk, rvec in vals:
                            plsc.store_scatter(tbuf.at[s], [fvk, bv], rvec)
                return carry

            lax.fori_loop(0, _CHUNK // _LANES // 2, kb_body, 0)

        def fire_stores(j, s):
            c = chunk0 + j
            h = c // btiles
            bb = lax.rem(c, btiles)
            for ft in range(ftiles):
                pltpu.async_copy(
                    tbuf.at[s, pl.ds(ft * 8, 8)],
                    out_hbm.at[h, ft, bb],
                    ssem.at[s],
                )

        fire_gather(0, 0)
        fire_gather(1, 1)

        def body(g, carry):
            for s in (0, 1):
                j = 2 * g + s
                drain_gather(s)

                @pl.when(g >= 1)
                def _():
                    wait_stores(s)

                transpose(s)

                @pl.when(g <= chunks_per_w // 2 - 2)
                def _():
                    fire_gather(j + 2, s)

                fire_stores(j, s)
            return carry

        lax.fori_loop(0, chunks_per_w // 2, body, 0)
        wait_stores(0)
        wait_stores(1)

    return gather_kernel


def kernel(idx, embedding):
    batch, hist = idx.shape
    feat = embedding.shape[1]
    idx1 = idx.T.reshape(-1).astype(jnp.int32)
    out5 = _build_gather(batch * hist, feat, hist)(idx1, embedding)
    return out5.transpose(2, 4, 0, 1, 3).reshape(batch, hist, feat)


# speedup vs baseline: 2.8573x; 1.0000x over previous
import functools

import jax
import jax.numpy as jnp
from jax import lax
from jax.experimental import pallas as pl
from jax.experimental.pallas import tpu as pltpu
from jax.experimental.pallas import tpu_sc as plsc

_NUM_CORES = 2
_NUM_SUBCORES = 16
_NUM_WORKERS = _NUM_CORES * _NUM_SUBCORES
_CHUNK = 128
_LANES = 16


@functools.cache
def _build_gather(num_rows: int, feat: int, hist: int):
    batch = num_rows // hist
    btiles = batch // _CHUNK
    ftiles = feat // 8
    chunks_total = num_rows // _CHUNK
    chunks_per_w = chunks_total // _NUM_WORKERS
    rows_per_w = chunks_per_w * _CHUNK

    mesh = plsc.VectorSubcoreMesh(core_axis_name="c", subcore_axis_name="s")

    @functools.partial(
        pl.kernel,
        mesh=mesh,
        out_type=jax.ShapeDtypeStruct(
            (hist, ftiles, btiles, 8, _CHUNK), jnp.float32
        ),
        scratch_types=[
            pltpu.VMEM((rows_per_w,), jnp.int32),
            pltpu.VMEM((2, _CHUNK, feat), jnp.float32),
            pltpu.VMEM((2, feat, _CHUNK), jnp.float32),
            pltpu.SemaphoreType.DMA((2,)),
            pltpu.SemaphoreType.DMA((2,)),
        ],
        compiler_params=pltpu.CompilerParams(
            use_tc_tiling_on_sc=False,
            needs_layout_passes=False,
            disable_bounds_checks=True,
        ),
    )
    def gather_kernel(idx_hbm, table_hbm, out_hbm, idx_v, rows, tbuf, gsem, ssem):
        wid = lax.axis_index("s") * _NUM_CORES + lax.axis_index("c")
        chunk0 = wid * chunks_per_w
        pltpu.sync_copy(idx_hbm.at[pl.ds(chunk0 * _CHUNK, rows_per_w)], idx_v)

        lane = lax.iota(jnp.int32, _LANES)
        fvecs = [lax.rem(lane + j, _LANES) for j in range(_LANES)]

        def fire_gather(j, s):
            pltpu.async_copy(
                table_hbm.at[idx_v.at[pl.ds(j * _CHUNK, _CHUNK)]],
                rows.at[s],
                gsem.at[s],
            )

        def drain_gather(s):
            pltpu.make_async_copy(
                table_hbm.at[pl.ds(0, _CHUNK)], rows.at[s], gsem.at[s]
            ).wait()

        def wait_stores(s):
            for ft in range(ftiles):
                pltpu.make_async_copy(
                    tbuf.at[s, pl.ds(ft * 8, 8)],
                    out_hbm.at[0, ft, 0],
                    ssem.at[s],
                ).wait()

        def transpose(s):
            def kb_body(kb, carry):
                for u in range(2):
                    bv = lane + (kb * 2 + u) * _LANES
                    for j0 in range(0, _LANES, 4):
                        vals = []
                        for j in range(j0, j0 + 4):
                            fv = fvecs[j]
                            for kf in range(feat // _LANES):
                                fvk = fv + _LANES * kf if kf else fv
                                vals.append(
                                    (fvk, plsc.load_gather(rows.at[s], [bv, fvk]))
                                )
                        for fvk, rvec in vals:
                            plsc.store_scatter(tbuf.at[s], [fvk, bv], rvec)
                return carry

            lax.fori_loop(0, _CHUNK // _LANES // 2, kb_body, 0)

        def fire_stores(j, s):
            c = chunk0 + j
            h = c // btiles
            bb = lax.rem(c, btiles)
            for ft in range(ftiles):
                pltpu.async_copy(
                    tbuf.at[s, pl.ds(ft * 8, 8)],
                    out_hbm.at[h, ft, bb],
                    ssem.at[s],
                )

        fire_gather(0, 0)
        fire_gather(1, 1)

        def body(g, carry):
            for s in (0, 1):
                j = 2 * g + s
                drain_gather(s)

                @pl.when(g >= 1)
                def _():
                    wait_stores(s)

                transpose(s)

                @pl.when(g <= chunks_per_w // 2 - 2)
                def _():
                    fire_gather(j + 2, s)

                fire_stores(j, s)
            return carry

        lax.fori_loop(0, chunks_per_w // 2, body, 0)
        wait_stores(0)
        wait_stores(1)

    return gather_kernel


def kernel(idx, embedding):
    batch, hist = idx.shape
    feat = embedding.shape[1]
    idx1 = idx.T.reshape(-1).astype(jnp.int32)
    out5 = _build_gather(batch * hist, feat, hist)(idx1, embedding)
    return out5.transpose(2, 4, 0, 1, 3).reshape(batch, hist, feat)
